# baseline (speedup 1.0000x reference)
import functools

import jax
import jax.numpy as jnp
from jax import lax
from jax.experimental import pallas as pl
from jax.experimental.pallas import tpu as pltpu
from jax.experimental.pallas import tpu_sc as plsc

BATCH = 4096
FIELDS = 26
EMB_DIM = 32
NUM_IDX = BATCH * FIELDS
PACK = 128 // EMB_DIM
SUPER_ROWS = 1000000 // PACK

NC = 2
NS = 16
NW = NC * NS
B_PER_W = NUM_IDX // NW
NCHUNK = 4
CHUNK = B_PER_W // NCHUNK
LANES = 16


def _make_gather():
  mesh = plsc.VectorSubcoreMesh(core_axis_name="c", subcore_axis_name="s")

  @functools.partial(
      pl.kernel,
      mesh=mesh,
      out_type=jax.ShapeDtypeStruct((NUM_IDX, 128), jnp.float32),
      scratch_types=[
          [pltpu.VMEM((CHUNK,), jnp.int32) for _ in range(NCHUNK)],
          pltpu.VMEM((B_PER_W,), jnp.int32),
          pltpu.VMEM((CHUNK, 128), jnp.float32),
          pltpu.SemaphoreType.DMA,
      ],
  )
  def gather(idx_hbm, table_hbm, out_hbm, sup_refs, idx_v, buf_v, sem):
    wid = lax.axis_index("s") * NC + lax.axis_index("c")
    base = wid * B_PER_W
    pltpu.sync_copy(idx_hbm.at[pl.ds(base, B_PER_W)], idx_v)

    for k in range(NCHUNK):
      sup_k = sup_refs[k]

      def mk_sup(o, _):
        v = idx_v[pl.ds((k * (CHUNK // LANES) + o) * LANES, LANES)]
        sup_k[pl.ds(o * LANES, LANES)] = lax.shift_right_logical(v, 2)
        return 0

      lax.fori_loop(0, CHUNK // LANES, mk_sup, 0, unroll=4)

    for k in range(NCHUNK):
      pltpu.async_copy(table_hbm.at[sup_refs[k]], buf_v, sem).wait()
      pltpu.sync_copy(buf_v, out_hbm.at[pl.ds(base + k * CHUNK, CHUNK)])

  return gather


_gather = _make_gather()


@jax.jit
def kernel(x, weight):
  idx = x.reshape(NUM_IDX).astype(jnp.int32)
  wt = weight.reshape(SUPER_ROWS, 128)
  sup = _gather(idx, wt)
  col = (idx % PACK) * EMB_DIM
  out = jax.vmap(lambda r, c: lax.dynamic_slice(r, (c,), (EMB_DIM,)))(sup, col)
  return out.reshape(BATCH, FIELDS, EMB_DIM)


# speedup vs baseline: 84.4554x; 84.4554x over previous
import functools

import jax
import jax.numpy as jnp
from jax import lax
from jax.experimental import pallas as pl
from jax.experimental.pallas import tpu as pltpu
from jax.experimental.pallas import tpu_sc as plsc

BATCH = 4096
FIELDS = 26
EMB_DIM = 32
NUM_IDX = BATCH * FIELDS
PACK = 128 // EMB_DIM
SUPER_ROWS = 1000000 // PACK

NC = 2
NS = 16
NW = NC * NS
B_PER_W = NUM_IDX // NW
NCHUNK = 4
CHUNK = B_PER_W // NCHUNK
LANES = 16


def _make_gather():
  mesh = plsc.VectorSubcoreMesh(core_axis_name="c", subcore_axis_name="s")

  @functools.partial(
      pl.kernel,
      mesh=mesh,
      out_type=jax.ShapeDtypeStruct((NUM_IDX, 128), jnp.float32),
      scratch_types=[
          [pltpu.VMEM((CHUNK,), jnp.int32) for _ in range(NCHUNK)],
          pltpu.VMEM((B_PER_W,), jnp.int32),
          pltpu.VMEM((CHUNK, 128), jnp.float32),
          pltpu.SemaphoreType.DMA,
      ],
  )
  def gather(idx_hbm, table_hbm, out_hbm, sup_refs, idx_v, buf_v, sem):
    wid = lax.axis_index("s") * NC + lax.axis_index("c")
    base = wid * B_PER_W
    pltpu.sync_copy(idx_hbm.at[pl.ds(base, B_PER_W)], idx_v)

    for k in range(NCHUNK):
      sup_k = sup_refs[k]

      def mk_sup(o, _):
        v = idx_v[pl.ds((k * (CHUNK // LANES) + o) * LANES, LANES)]
        sup_k[pl.ds(o * LANES, LANES)] = lax.shift_right_logical(v, 2)
        return 0

      lax.fori_loop(0, CHUNK // LANES, mk_sup, 0, unroll=4)

    for k in range(NCHUNK):
      pltpu.async_copy(table_hbm.at[sup_refs[k]], buf_v, sem).wait()
      pltpu.sync_copy(buf_v, out_hbm.at[pl.ds(base + k * CHUNK, CHUNK)])

  return gather


_gather = _make_gather()


@jax.jit
def kernel(x, weight):
  idx = x.reshape(NUM_IDX).astype(jnp.int32)
  wt = weight.reshape(SUPER_ROWS, 128)
  sup = _gather(idx, wt)
  s4 = sup.reshape(NUM_IDX, PACK, EMB_DIM)
  q = (idx & (PACK - 1))[:, None]
  out = jnp.where(
      q == 0, s4[:, 0],
      jnp.where(q == 1, s4[:, 1], jnp.where(q == 2, s4[:, 2], s4[:, 3])))
  return out.reshape(BATCH, FIELDS, EMB_DIM)


# speedup vs baseline: 241.0382x; 2.8540x over previous
import functools

import jax
import jax.numpy as jnp
from jax import lax
from jax.experimental import pallas as pl
from jax.experimental.pallas import tpu as pltpu
from jax.experimental.pallas import tpu_sc as plsc

BATCH = 4096
FIELDS = 26
EMB_DIM = 32
NUM_IDX = BATCH * FIELDS

NC = 2
NS = 16
NW = NC * NS
B_PER_W = NUM_IDX // NW
NCHUNK = 8
CHUNK = B_PER_W // NCHUNK


def _make_gather():
  mesh = plsc.VectorSubcoreMesh(core_axis_name="c", subcore_axis_name="s")

  @functools.partial(
      pl.kernel,
      mesh=mesh,
      out_type=jax.ShapeDtypeStruct((NUM_IDX, EMB_DIM), jnp.float32),
      scratch_types=[
          pltpu.VMEM((B_PER_W,), jnp.int32),
          [pltpu.VMEM((CHUNK, EMB_DIM), jnp.float32) for _ in range(2)],
          [pltpu.SemaphoreType.DMA for _ in range(2)],
          [pltpu.SemaphoreType.DMA for _ in range(2)],
      ],
  )
  def gather(idx_hbm, table_hbm, out_hbm, idx_v, bufs, insems, outsems):
    wid = lax.axis_index("s") * NC + lax.axis_index("c")
    base = wid * B_PER_W
    pltpu.sync_copy(idx_hbm.at[pl.ds(base, B_PER_W)], idx_v)

    def issue_chunk(c, buf, insem):
      def body(g, _):
        v = idx_v[pl.ds(c * CHUNK + g * 16, 16)]
        for j in range(16):
          pltpu.async_copy(
              table_hbm.at[pl.ds(v[j], 1)],
              buf.at[pl.ds(g * 16 + j, 1)], insem)
        return 0

      lax.fori_loop(0, CHUNK // 16, body, 0)

    def drain_chunk(buf, insem):
      def body(r, _):
        pltpu.make_async_copy(
            table_hbm.at[pl.ds(0, 1)], buf.at[pl.ds(0, 1)], insem).wait()
        return 0

      lax.fori_loop(0, CHUNK, body, 0, unroll=8)

    for c in range(NCHUNK + 1):
      if c < NCHUNK:
        p = c % 2
        if c >= 2:
          pltpu.make_async_copy(
              bufs[p], out_hbm.at[pl.ds(base, CHUNK)], outsems[p]).wait()
        issue_chunk(c, bufs[p], insems[p])
      if c >= 1:
        q = (c - 1) % 2
        drain_chunk(bufs[q], insems[q])
        pltpu.async_copy(
            bufs[q], out_hbm.at[pl.ds(base + (c - 1) * CHUNK, CHUNK)],
            outsems[q])
    for c in (NCHUNK - 2, NCHUNK - 1):
      p = c % 2
      pltpu.make_async_copy(
          bufs[p], out_hbm.at[pl.ds(base, CHUNK)], outsems[p]).wait()

  return gather


_gather = _make_gather()


@jax.jit
def kernel(x, weight):
  idx = x.reshape(NUM_IDX).astype(jnp.int32)
  out = _gather(idx, weight)
  return out.reshape(BATCH, FIELDS, EMB_DIM)


# speedup vs baseline: 258.7972x; 1.0737x over previous
import functools

import jax
import jax.numpy as jnp
from jax import lax
from jax.experimental import pallas as pl
from jax.experimental.pallas import tpu as pltpu
from jax.experimental.pallas import tpu_sc as plsc

BATCH = 4096
FIELDS = 26
EMB_DIM = 32

NC = 2
NS = 16
NW = NC * NS
BATCH_PER_W = BATCH // NW
CB = 8
NCHUNK = BATCH_PER_W // CB


def _make_lookup():
  mesh = plsc.VectorSubcoreMesh(core_axis_name="c", subcore_axis_name="s")

  @functools.partial(
      pl.kernel,
      mesh=mesh,
      out_type=jax.ShapeDtypeStruct((BATCH, FIELDS, EMB_DIM), jnp.float32),
      scratch_types=[
          pltpu.VMEM((BATCH_PER_W * FIELDS,), jnp.int32),
          [pltpu.VMEM((CB, FIELDS, EMB_DIM), jnp.float32) for _ in range(2)],
          [pltpu.SemaphoreType.DMA for _ in range(2)],
          [pltpu.SemaphoreType.DMA for _ in range(2)],
      ],
  )
  def lookup(idx_hbm, table_hbm, out_hbm, idx_v, bufs, insems, outsems):
    wid = lax.axis_index("s") * NC + lax.axis_index("c")
    base = wid * BATCH_PER_W
    pltpu.sync_copy(
        idx_hbm.at[pl.ds(base * FIELDS, BATCH_PER_W * FIELDS)], idx_v)

    def issue_chunk(c, buf, insem):
      def body(g, _):
        v = idx_v[pl.ds(c * (CB * FIELDS) + g * 16, 16)]
        for j in range(16):
          p = g * 16 + j
          bb = p // FIELDS
          f = p - bb * FIELDS
          pltpu.async_copy(
              table_hbm.at[pl.ds(v[j], 1)],
              buf.at[bb, pl.ds(f, 1)], insem)
        return 0

      lax.fori_loop(0, CB * FIELDS // 16, body, 0)

    def drain_chunk(buf, insem):
      def body(r, _):
        pltpu.make_async_copy(
            table_hbm.at[pl.ds(0, 1)],
            buf.at[0, pl.ds(0, 1)], insem).wait()
        return 0

      lax.fori_loop(0, CB * FIELDS, body, 0, unroll=8)

    for c in range(NCHUNK + 1):
      if c < NCHUNK:
        p = c % 2
        if c >= 2:
          pltpu.make_async_copy(
              bufs[p], out_hbm.at[pl.ds(base, CB)], outsems[p]).wait()
        issue_chunk(c, bufs[p], insems[p])
      if c >= 1:
        q = (c - 1) % 2
        drain_chunk(bufs[q], insems[q])
        pltpu.async_copy(
            bufs[q], out_hbm.at[pl.ds(base + (c - 1) * CB, CB)], outsems[q])
    for c in (NCHUNK - 2, NCHUNK - 1):
      p = c % 2
      pltpu.make_async_copy(
          bufs[p], out_hbm.at[pl.ds(base, CB)], outsems[p]).wait()

  return lookup


_lookup = _make_lookup()


@jax.jit
def kernel(x, weight):
  idx = x.reshape(BATCH * FIELDS).astype(jnp.int32)
  return _lookup(idx, weight)
